# baseline (speedup 1.0000x reference)
import functools

import jax
import jax.numpy as jnp
from jax import lax
from jax.experimental import pallas as pl
from jax.experimental.pallas import tpu as pltpu
from jax.experimental.pallas import tpu_sc as plsc

B = 16384
D = 64
CH = 128


def _pack_body(tab, o, bufa, bufb, sema, semb):
    i = pl.program_id(0)
    rb = bufa.shape[0]
    half = tab.shape[0] // 2
    ca = pltpu.make_async_copy(tab.at[pl.ds(i * rb, rb)], bufa, sema)
    cb = pltpu.make_async_copy(tab.at[pl.ds(half + i * rb, rb)], bufb, semb)
    ca.start()
    cb.start()
    ca.wait()
    cb.wait()
    o[:, pl.ds(0, D)] = bufa[...]
    o[:, pl.ds(D, D)] = bufb[...]


def _tc_pack(tab, rb):
    n = tab.shape[0]
    return pl.pallas_call(
        _pack_body,
        grid=(n // 2 // rb,),
        in_specs=[pl.BlockSpec(memory_space=pl.ANY)],
        out_specs=pl.BlockSpec((rb, 2 * D), lambda i: (i, 0)),
        out_shape=jax.ShapeDtypeStruct((n // 2, 2 * D), jnp.float32),
        scratch_shapes=[
            pltpu.VMEM((rb, D), jnp.float32),
            pltpu.VMEM((rb, D), jnp.float32),
            pltpu.SemaphoreType.DMA,
            pltpu.SemaphoreType.DMA,
        ],
    )(tab)


def _sc_gather(Ppk, Qpk, vP, vQ, offP, offQ):
    info = plsc.get_sparse_core_info()
    NC, NS, L = info.num_cores, info.num_subcores, info.num_lanes
    NW = NC * NS
    bpw = B // NW
    hb = bpw // 2
    nch = hb // CH
    mesh = plsc.VectorSubcoreMesh(core_axis_name="c", subcore_axis_name="s")

    vP3 = vP.reshape(NW, 2 * nch, CH)
    vQ3 = vQ.reshape(NW, 2 * nch, CH)
    oP2 = offP.reshape(NW, bpw)
    oQ2 = offQ.reshape(NW, bpw)

    @functools.partial(
        pl.kernel,
        mesh=mesh,
        out_type=jax.ShapeDtypeStruct((B, 2 * D), jnp.float32),
        scratch_types=[
            pltpu.VMEM((2 * nch, CH), jnp.int32),
            pltpu.VMEM((2 * nch, CH), jnp.int32),
            pltpu.VMEM((bpw,), jnp.int32),
            pltpu.VMEM((bpw,), jnp.int32),
            pltpu.VMEM((hb, 2 * D), jnp.float32),
            pltpu.VMEM((bpw, 2 * D), jnp.float32),
            pltpu.SemaphoreType.DMA,
        ],
    )
    def k(P_hbm, Q_hbm, vp_hbm, vq_hbm, op_hbm, oq_hbm, Xout,
          vp, vq, op, oq, Gbuf, Xv, sem):
        wid = lax.axis_index("s") * NC + lax.axis_index("c")
        base = wid * bpw
        pltpu.sync_copy(vp_hbm.at[wid], vp)
        pltpu.sync_copy(vq_hbm.at[wid], vq)
        pltpu.sync_copy(op_hbm.at[wid], op)
        pltpu.sync_copy(oq_hbm.at[wid], oq)

        for src, idx, offv, col0 in (
            (P_hbm, vp, op, 0),
            (Q_hbm, vq, oq, D),
        ):
            for h in range(2):
                copies = []
                for c in range(nch):
                    copies.append(
                        pltpu.async_copy(
                            src.at[idx.at[h * nch + c]],
                            Gbuf.at[pl.ds(c * CH, CH)],
                            sem,
                        )
                    )
                for cp in copies:
                    cp.wait()

                def body(t, _):
                    offvec = offv[pl.ds(h * hb + t * L, L)]
                    for l in range(L):
                        j = t * L + l
                        off = offvec[l]
                        row = h * hb + j
                        for q in range(D // L):
                            Xv[row, pl.ds(col0 + q * L, L)] = Gbuf[
                                j, pl.ds(off + q * L, L)
                            ]
                    return 0

                lax.fori_loop(0, hb // L, body, 0)

        pltpu.sync_copy(Xv, Xout.at[pl.ds(base, bpw)])

    return k(Ppk, Qpk, vP3, vQ3, oP2, oQ2)


def _mlp_body(x, w1, b1, w2, b2, o):
    h = jnp.dot(x[...], w1[...], preferred_element_type=jnp.float32)
    h = jnp.maximum(h + b1[...], 0.0)
    o[...] = jnp.sum(h * w2[...], axis=1, keepdims=True) + b2[...]


def _tc_mlp(X, W1, b1r, w2r, b2r):
    TB = 2048
    return pl.pallas_call(
        _mlp_body,
        grid=(B // TB,),
        in_specs=[
            pl.BlockSpec((TB, 2 * D), lambda i: (i, 0)),
            pl.BlockSpec((2 * D, D), lambda i: (0, 0)),
            pl.BlockSpec((1, D), lambda i: (0, 0)),
            pl.BlockSpec((1, D), lambda i: (0, 0)),
            pl.BlockSpec((1, 1), lambda i: (0, 0)),
        ],
        out_specs=pl.BlockSpec((TB, 1), lambda i: (i, 0)),
        out_shape=jax.ShapeDtypeStruct((B, 1), jnp.float32),
    )(X, W1, b1r, w2r, b2r)


def kernel(user, product, P_table, Q_table, W1, b1, W2, b2):
    user = user.astype(jnp.int32)
    product = product.astype(jnp.int32)
    halfP = P_table.shape[0] // 2
    halfQ = Q_table.shape[0] // 2
    Ppk = _tc_pack(P_table, 20000)
    Qpk = _tc_pack(Q_table, 10000)
    vP = user % halfP
    offP = (user // halfP) * D
    vQ = product % halfQ
    offQ = (product // halfQ) * D
    X = _sc_gather(Ppk, Qpk, vP, vQ, offP, offQ)
    return _tc_mlp(
        X,
        W1,
        b1.reshape(1, D),
        W2.reshape(1, D),
        b2.reshape(1, 1),
    )


# speedup vs baseline: 1.2881x; 1.2881x over previous
import functools

import jax
import jax.numpy as jnp
from jax import lax
from jax.experimental import pallas as pl
from jax.experimental.pallas import tpu as pltpu
from jax.experimental.pallas import tpu_sc as plsc

B = 16384
D = 64
CH = 128


def _sc_gather(Pp, Qp, uidx, pidx):
    info = plsc.get_sparse_core_info()
    NC, NS, L = info.num_cores, info.num_subcores, info.num_lanes
    NW = NC * NS
    bpw = B // NW
    nch = bpw // CH
    mesh = plsc.VectorSubcoreMesh(core_axis_name="c", subcore_axis_name="s")

    u3 = uidx.reshape(NW, nch, CH)
    p3 = pidx.reshape(NW, nch, CH)

    @functools.partial(
        pl.kernel,
        mesh=mesh,
        compiler_params=pltpu.CompilerParams(use_tc_tiling_on_sc=False),
        out_type=[
            jax.ShapeDtypeStruct((B, 2 * D), jnp.float32),
            jax.ShapeDtypeStruct((B, 2 * D), jnp.float32),
        ],
        scratch_types=[
            pltpu.VMEM((nch, CH), jnp.int32),
            pltpu.VMEM((nch, CH), jnp.int32),
            pltpu.VMEM((bpw, 2 * D), jnp.float32),
            pltpu.SemaphoreType.DMA,
        ],
    )
    def k(P_hbm, Q_hbm, u_hbm, pr_hbm, Pout, Qout, uv, pv, buf, sem):
        wid = lax.axis_index("s") * NC + lax.axis_index("c")
        base = wid * bpw
        pltpu.sync_copy(u_hbm.at[wid], uv)
        pltpu.sync_copy(pr_hbm.at[wid], pv)
        for idx, src, out in ((uv, P_hbm, Pout), (pv, Q_hbm, Qout)):
            copies = []
            for c in range(nch):
                copies.append(
                    pltpu.async_copy(
                        src.at[idx.at[c]], buf.at[pl.ds(c * CH, CH)], sem
                    )
                )
            for cp in copies:
                cp.wait()
            pltpu.sync_copy(buf, out.at[pl.ds(base, bpw)])

    return k(Pp, Qp, u3, p3)


def _mlp_body(p, q, w1a, w1b, b1, w2, b2, o):
    h = jnp.dot(p[...], w1a[...], preferred_element_type=jnp.float32)
    h = h + jnp.dot(q[...], w1b[...], preferred_element_type=jnp.float32)
    h = jnp.maximum(h + b1[...], 0.0)
    o[...] = jnp.sum(h * w2[...], axis=1, keepdims=True) + b2[...]


def _tc_mlp(P, Q, W1a, W1b, b1r, w2r, b2r):
    TB = 2048
    return pl.pallas_call(
        _mlp_body,
        grid=(B // TB,),
        in_specs=[
            pl.BlockSpec((TB, 2 * D), lambda i: (i, 0)),
            pl.BlockSpec((TB, 2 * D), lambda i: (i, 0)),
            pl.BlockSpec((2 * D, D), lambda i: (0, 0)),
            pl.BlockSpec((2 * D, D), lambda i: (0, 0)),
            pl.BlockSpec((1, D), lambda i: (0, 0)),
            pl.BlockSpec((1, D), lambda i: (0, 0)),
            pl.BlockSpec((1, 1), lambda i: (0, 0)),
        ],
        out_specs=pl.BlockSpec((TB, 1), lambda i: (i, 0)),
        out_shape=jax.ShapeDtypeStruct((B, 1), jnp.float32),
    )(P, Q, W1a, W1b, b1r, w2r, b2r)


def kernel(user, product, P_table, Q_table, W1, b1, W2, b2):
    user = user.astype(jnp.int32)
    product = product.astype(jnp.int32)
    Pp = jnp.pad(P_table, ((0, 0), (0, D)))
    Qp = jnp.pad(Q_table, ((0, 0), (0, D)))
    P, Q = _sc_gather(Pp, Qp, user, product)
    Z = jnp.zeros((D, D), jnp.float32)
    W1a = jnp.concatenate([W1[:D], Z], axis=0)
    W1b = jnp.concatenate([W1[D:], Z], axis=0)
    return _tc_mlp(
        P,
        Q,
        W1a,
        W1b,
        b1.reshape(1, D),
        W2.reshape(1, D),
        b2.reshape(1, 1),
    )
